# baseline (speedup 1.0000x reference)
import functools

import jax
import jax.numpy as jnp
from jax import lax
from jax.experimental import pallas as pl
from jax.experimental.pallas import tpu as pltpu
from jax.experimental.pallas import tpu_sc as plsc

_MUL = 16
_NIR = 3
_DIM = 9
_KIDX = (0, 1, 1, 1, 2, 2, 2, 2, 2)
_OUTW = _MUL * _DIM
_WW = _MUL * _NIR
_LANES = 16
_NC = 2
_NS = 16
_NW = _NC * _NS
_CHUNK = 200


def _sc_body(n_chunks, a_hbm, w_hbm, o_hbm,
             a_v0, a_v1, w_v0, w_v1, o_v0, o_v1,
             sa0, sa1, sw0, sw1, so0, so1):
  wid = lax.axis_index("s") * _NC + lax.axis_index("c")
  base = wid * n_chunks * _CHUNK
  A, W, O = (a_v0, a_v1), (w_v0, w_v1), (o_v0, o_v1)
  SA, SW, SO = (sa0, sa1), (sw0, sw1), (so0, so1)

  def in_copies(t, b):
    row0 = base + t * _CHUNK
    return (
        pltpu.make_async_copy(
            a_hbm.at[pl.ds(row0 * _DIM, _CHUNK * _DIM)],
            A[b].at[pl.ds(0, _CHUNK * _DIM)], SA[b]),
        pltpu.make_async_copy(
            w_hbm.at[pl.ds(row0 * _WW, _CHUNK * _WW)], W[b], SW[b]),
    )

  def out_copy(t, b):
    row0 = base + t * _CHUNK
    return pltpu.make_async_copy(
        O[b], o_hbm.at[pl.ds(row0 * _OUTW, _CHUNK * _OUTW)], SO[b])

  def start_in(t, b):
    for c in in_copies(t, b):
      c.start()

  def wait_in(t, b):
    for c in in_copies(t, b):
      c.wait()

  lane = lax.iota(jnp.int32, _LANES)
  l3 = lane * _NIR
  l9 = lane * _DIM

  def compute(b):
    a_v, w_v, o_v = A[b], W[b], O[b]

    def row(r, c):
      ab = r * _DIM
      wb = r * _WW
      ob = r * _OUTW
      av16 = a_v[pl.ds(ab, _LANES)]
      for dd in range(_DIM):
        wv = plsc.load_gather(w_v, [l3 + (wb + _KIDX[dd])])
        plsc.store_scatter(o_v, [l9 + (ob + dd)], wv * av16[dd])
      return c

    lax.fori_loop(0, _CHUNK, row, 0)

  start_in(0, 0)
  start_in(1, 1)
  for t in (0, 1):
    wait_in(t, t & 1)
    compute(t & 1)
    out_copy(t, t & 1).start()
    start_in(t + 2, t & 1)

  def main_body(k, carry):
    t0 = 2 + 2 * k
    for b in (0, 1):
      t = t0 + b
      wait_in(t, b)
      out_copy(t - 2, b).wait()
      compute(b)
      out_copy(t, b).start()
      start_in(t + 2, b)
    return carry

  lax.fori_loop(0, (n_chunks - 4) // 2, main_body, 0)

  for t in (n_chunks - 2, n_chunks - 1):
    b = t & 1
    wait_in(t, b)
    out_copy(t - 2, b).wait()
    compute(b)
    out_copy(t, b).start()
  out_copy(n_chunks - 2, 0).wait()
  out_copy(n_chunks - 1, 1).wait()


@jax.jit
def _run(a1d, w1d):
  e_total = a1d.shape[0] // _DIM
  n_chunks = e_total // (_NW * _CHUNK)
  mesh = plsc.VectorSubcoreMesh(core_axis_name="c", subcore_axis_name="s")
  body = functools.partial(_sc_body, n_chunks)
  sc_kernel = pl.kernel(
      body,
      out_type=jax.ShapeDtypeStruct((e_total * _OUTW,), jnp.float32),
      mesh=mesh,
      compiler_params=pltpu.CompilerParams(needs_layout_passes=False),
      scratch_types=(
          [pltpu.VMEM((_CHUNK * _DIM + _LANES,), jnp.float32)] * 2
          + [pltpu.VMEM((_CHUNK * _WW,), jnp.float32)] * 2
          + [pltpu.VMEM((_CHUNK * _OUTW,), jnp.float32)] * 2
          + [pltpu.SemaphoreType.DMA] * 6
      ),
  )
  return sc_kernel(a1d, w1d)


def kernel(edge_attr, weights):
  e = edge_attr.shape[0]
  assert e % (_NW * _CHUNK) == 0 and e // (_NW * _CHUNK) >= 6, e
  out = _run(edge_attr.reshape(-1), weights.reshape(-1))
  return out.reshape(e, _MUL, _DIM)
